# Initial kernel scaffold; baseline (speedup 1.0000x reference)
import jax
import jax.numpy as jnp
from jax.experimental import pallas as pl


def kernel(x, tables, W0, b0, g0, bb0, W1, b1, g1, bb1, W2, b2, g2, bb2, W3, b3):
    raise NotImplementedError("write your pallas kernel here")


# speedup vs baseline: 14.5957x; 14.5957x over previous
import functools

import numpy as np
import jax
import jax.numpy as jnp
from jax import lax
from jax.experimental import pallas as pl
from jax.experimental.pallas import tpu as pltpu
from jax.experimental.pallas import tpu_sc as plsc

_F = 26
_ROWS = 4000
_TOT = _F * _ROWS
_D = 16
_B = 4096
_NP = _F * (_F - 1) // 2
_BN_S = float(1.0 / np.sqrt(1.0 + 1e-5))

_OFF = np.arange(_F, dtype=np.int64) * _ROWS
_I, _J = np.triu_indices(_F, k=1)
_RAW_CONST = ((_F - 1) * _TOT + _OFF).astype(np.int32)
_A_CONST = ((_J - 1) * _TOT + _OFF[_I]).astype(np.int32)
_B_CONST = (_I * _TOT + _OFF[_J]).astype(np.int32)

_CHUNK = 1664


def _sc_gather(table_flat, raw_idx, a_idx, b_idx):
    info = plsc.get_sparse_core_info()
    nc, ns = info.num_cores, info.num_subcores
    nw = nc * ns
    mesh = plsc.VectorSubcoreMesh(core_axis_name="c", subcore_axis_name="s")
    out_type = (
        jax.ShapeDtypeStruct((_B * _F, _D), jnp.float32),
        jax.ShapeDtypeStruct((_B * _NP, _D), jnp.float32),
        jax.ShapeDtypeStruct((_B * _NP, _D), jnp.float32),
    )

    @functools.partial(
        pl.kernel,
        mesh=mesh,
        out_type=out_type,
        compiler_params=pltpu.CompilerParams(use_tc_tiling_on_sc=False),
        scratch_types=[
            pltpu.VMEM((_CHUNK,), jnp.int32),
            pltpu.VMEM((_CHUNK, _D), jnp.float32),
            pltpu.SemaphoreType.DMA,
        ],
    )
    def k(table_hbm, ri_hbm, ai_hbm, bi_hbm, ro_hbm, ao_hbm, bo_hbm,
          idx_v, rows_v, sem):
        wid = lax.axis_index("s") * nc + lax.axis_index("c")

        def section(idx_hbm, out_hbm, n_rows):
            per_w = n_rows // nw
            base = wid * per_w

            def body(i, carry):
                off = base + i * _CHUNK
                pltpu.sync_copy(idx_hbm.at[pl.ds(off, _CHUNK)], idx_v)
                pltpu.async_copy(table_hbm.at[idx_v], rows_v, sem).wait()
                pltpu.sync_copy(rows_v, out_hbm.at[pl.ds(off, _CHUNK)])
                return carry

            lax.fori_loop(0, per_w // _CHUNK, body, 0)

        section(ri_hbm, ro_hbm, _B * _F)
        section(ai_hbm, ao_hbm, _B * _NP)
        section(bi_hbm, bo_hbm, _B * _NP)

    return k(table_flat, raw_idx, a_idx, b_idx)


def _tc_body(raw_ref, a_ref, b_ref, w0t_ref, w0r_ref, b0_ref, w1_ref, b1_ref,
             w2_ref, b2_ref, w3_ref, b3_ref, out_ref):
    p = a_ref[...] * b_ref[...]
    h = jnp.dot(raw_ref[...], w0t_ref[...], preferred_element_type=jnp.float32)
    h = h + jnp.dot(p, w0r_ref[...], preferred_element_type=jnp.float32)
    h = jnp.maximum(h + b0_ref[...], 0.0)
    h = jnp.dot(h, w1_ref[...], preferred_element_type=jnp.float32)
    h = jnp.maximum(h + b1_ref[...], 0.0)
    h = jnp.dot(h, w2_ref[...], preferred_element_type=jnp.float32)
    h = jnp.maximum(h + b2_ref[...], 0.0)
    o = jnp.dot(h, w3_ref[...], preferred_element_type=jnp.float32)
    out_ref[...] = jax.nn.sigmoid(o + b3_ref[...])


def _tc_mlp(raw, a, b, w0t, w0r, b0, w1, b1, w2, b2, w3, b3):
    blk = 256
    grid = (_B // blk,)

    def full(arr):
        return pl.BlockSpec(arr.shape, lambda i: (0,) * arr.ndim)

    return pl.pallas_call(
        _tc_body,
        grid=grid,
        in_specs=[
            pl.BlockSpec((blk, _F * _D), lambda i: (i, 0)),
            pl.BlockSpec((blk, _NP * _D), lambda i: (i, 0)),
            pl.BlockSpec((blk, _NP * _D), lambda i: (i, 0)),
            full(w0t), full(w0r), full(b0), full(w1), full(b1),
            full(w2), full(b2), full(w3), full(b3),
        ],
        out_specs=pl.BlockSpec((blk, 1), lambda i: (i, 0)),
        out_shape=jax.ShapeDtypeStruct((_B, 1), jnp.float32),
    )(raw, a, b, w0t, w0r, b0, w1, b1, w2, b2, w3, b3)


def kernel(x, tables, W0, b0, g0, bb0, W1, b1, g1, bb1, W2, b2, g2, bb2,
           W3, b3):
    table_flat = tables.reshape(_F * _TOT, _D)

    raw_idx = (x + jnp.asarray(_RAW_CONST)[None, :]).reshape(-1)
    a_idx = (x[:, jnp.asarray(_I.astype(np.int32))]
             + jnp.asarray(_A_CONST)[None, :]).reshape(-1)
    b_idx = (x[:, jnp.asarray(_J.astype(np.int32))]
             + jnp.asarray(_B_CONST)[None, :]).reshape(-1)

    raw_r, a_r, b_r = _sc_gather(table_flat, raw_idx, a_idx, b_idx)
    raw2 = raw_r.reshape(_B, _F * _D)
    a2 = a_r.reshape(_B, _NP * _D)
    b2_ = b_r.reshape(_B, _NP * _D)

    def fold(w, bias, g, bb):
        s = g * _BN_S
        return w * s[None, :], bias * s + bb

    w0f, b0f = fold(W0, b0, g0, bb0)
    w1f, b1f = fold(W1, b1, g1, bb1)
    w2f, b2f = fold(W2, b2, g2, bb2)
    w0t = w0f[: _F * _D]
    w0r = jnp.repeat(w0f[_F * _D:], _D, axis=0)

    out = _tc_mlp(raw2, a2, b2_, w0t, w0r, b0f[None, :], w1f, b1f[None, :],
                  w2f, b2f[None, :], W3, b3[None, :])
    return out.reshape(_B)


# speedup vs baseline: 14.7806x; 1.0127x over previous
import functools

import numpy as np
import jax
import jax.numpy as jnp
from jax import lax
from jax.experimental import pallas as pl
from jax.experimental.pallas import tpu as pltpu
from jax.experimental.pallas import tpu_sc as plsc

_F = 26
_ROWS = 4000
_TOT = _F * _ROWS
_D = 16
_B = 4096
_NP = _F * (_F - 1) // 2
_BN_S = float(1.0 / np.sqrt(1.0 + 1e-5))

_OFF = np.arange(_F, dtype=np.int64) * _ROWS
_I, _J = np.triu_indices(_F, k=1)
_RAW_CONST = ((_F - 1) * _TOT + _OFF).astype(np.int32)
_A_CONST = ((_J - 1) * _TOT + _OFF[_I]).astype(np.int32)
_B_CONST = (_I * _TOT + _OFF[_J]).astype(np.int32)
_ONEHOT_I = np.zeros((_F, _NP), np.float32)
_ONEHOT_I[_I, np.arange(_NP)] = 1.0
_ONEHOT_J = np.zeros((_F, _NP), np.float32)
_ONEHOT_J[_J, np.arange(_NP)] = 1.0

_RAW_CHUNK = 1664
_PAIR_CHUNK = 1664


def _sc_gather_inter(table_flat, raw_idx, a_idx, b_idx):
    info = plsc.get_sparse_core_info()
    nc, ns = info.num_cores, info.num_subcores
    nw = nc * ns
    mesh = plsc.VectorSubcoreMesh(core_axis_name="c", subcore_axis_name="s")
    out_type = (
        jax.ShapeDtypeStruct((_B * _F, _D), jnp.float32),
        jax.ShapeDtypeStruct((_B * _NP,), jnp.float32),
    )

    @functools.partial(
        pl.kernel,
        mesh=mesh,
        out_type=out_type,
        compiler_params=pltpu.CompilerParams(use_tc_tiling_on_sc=False,
                                             needs_layout_passes=False),
        scratch_types=[
            pltpu.VMEM((_PAIR_CHUNK,), jnp.int32),
            pltpu.VMEM((_PAIR_CHUNK,), jnp.int32),
            pltpu.VMEM((_PAIR_CHUNK, _D), jnp.float32),
            pltpu.VMEM((_PAIR_CHUNK, _D), jnp.float32),
            pltpu.VMEM((_PAIR_CHUNK,), jnp.float32),
            pltpu.VMEM((_RAW_CHUNK,), jnp.int32),
            pltpu.VMEM((_RAW_CHUNK, _D), jnp.float32),
            pltpu.SemaphoreType.DMA,
        ],
    )
    def k(table_hbm, ri_hbm, ai_hbm, bi_hbm, ro_hbm, io_hbm,
          ai_v, bi_v, ar_v, br_v, int_v, ri_v, rr_v, sem):
        wid = lax.axis_index("s") * nc + lax.axis_index("c")

        per_raw = _B * _F // nw
        rbase = wid * per_raw

        def rbody(i, c):
            off = rbase + i * _RAW_CHUNK
            pltpu.sync_copy(ri_hbm.at[pl.ds(off, _RAW_CHUNK)], ri_v)
            pltpu.async_copy(table_hbm.at[ri_v], rr_v, sem).wait()
            pltpu.sync_copy(rr_v, ro_hbm.at[pl.ds(off, _RAW_CHUNK)])
            return c

        lax.fori_loop(0, per_raw // _RAW_CHUNK, rbody, 0)

        per_pair = _B * _NP // nw
        pbase = wid * per_pair
        nblk = _PAIR_CHUNK // 16
        lanes = lax.iota(jnp.int32, 16)

        def pbody(c, carry):
            off = pbase + c * _PAIR_CHUNK
            pltpu.sync_copy(ai_hbm.at[pl.ds(off, _PAIR_CHUNK)], ai_v)
            pltpu.sync_copy(bi_hbm.at[pl.ds(off, _PAIR_CHUNK)], bi_v)
            ca = pltpu.async_copy(table_hbm.at[ai_v], ar_v, sem)
            cb = pltpu.async_copy(table_hbm.at[bi_v], br_v, sem)
            ca.wait()
            cb.wait()

            def dblk(k2, c2):
                rows = k2 * 16 + lanes
                acc = jnp.zeros((16,), jnp.float32)
                for d in range(_D):
                    cols = jnp.full((16,), d, jnp.int32)
                    va = plsc.load_gather(ar_v, [rows, cols])
                    vb = plsc.load_gather(br_v, [rows, cols])
                    acc = acc + va * vb
                int_v[pl.ds(k2 * 16, 16)] = acc
                return c2

            lax.fori_loop(0, nblk, dblk, 0)
            pltpu.sync_copy(int_v, io_hbm.at[pl.ds(off, _PAIR_CHUNK)])
            return carry

        lax.fori_loop(0, per_pair // _PAIR_CHUNK, pbody, 0)

    return k(table_flat, raw_idx, a_idx, b_idx)


def _tc_body(raw_ref, int_ref, w0t_ref, w0b_ref, b0_ref, w1_ref, b1_ref,
             w2_ref, b2_ref, w3_ref, b3_ref, out_ref):
    h = jnp.dot(raw_ref[...], w0t_ref[...], preferred_element_type=jnp.float32)
    h = h + jnp.dot(int_ref[...], w0b_ref[...],
                    preferred_element_type=jnp.float32)
    h = jnp.maximum(h + b0_ref[...], 0.0)
    h = jnp.dot(h, w1_ref[...], preferred_element_type=jnp.float32)
    h = jnp.maximum(h + b1_ref[...], 0.0)
    h = jnp.dot(h, w2_ref[...], preferred_element_type=jnp.float32)
    h = jnp.maximum(h + b2_ref[...], 0.0)
    o = jnp.dot(h, w3_ref[...], preferred_element_type=jnp.float32)
    out_ref[...] = jax.nn.sigmoid(o + b3_ref[...])


def _tc_mlp(raw, inter, w0t, w0b, b0, w1, b1, w2, b2, w3, b3):
    blk = 256
    grid = (_B // blk,)

    def full(arr):
        return pl.BlockSpec(arr.shape, lambda i: (0,) * arr.ndim)

    return pl.pallas_call(
        _tc_body,
        grid=grid,
        in_specs=[
            pl.BlockSpec((blk, _F * _D), lambda i: (i, 0)),
            pl.BlockSpec((blk, _NP), lambda i: (i, 0)),
            full(w0t), full(w0b), full(b0), full(w1), full(b1),
            full(w2), full(b2), full(w3), full(b3),
        ],
        out_specs=pl.BlockSpec((blk, 1), lambda i: (i, 0)),
        out_shape=jax.ShapeDtypeStruct((_B, 1), jnp.float32),
    )(raw, inter, w0t, w0b, b0, w1, b1, w2, b2, w3, b3)


def kernel(x, tables, W0, b0, g0, bb0, W1, b1, g1, bb1, W2, b2, g2, bb2,
           W3, b3):
    table_flat = tables.reshape(_F * _TOT, _D)

    raw_idx = (x + jnp.asarray(_RAW_CONST)[None, :]).reshape(-1)
    xf = x.astype(jnp.float32)
    a_idx = (jnp.dot(xf, jnp.asarray(_ONEHOT_I)).astype(jnp.int32)
             + jnp.asarray(_A_CONST)[None, :]).reshape(-1)
    b_idx = (jnp.dot(xf, jnp.asarray(_ONEHOT_J)).astype(jnp.int32)
             + jnp.asarray(_B_CONST)[None, :]).reshape(-1)

    raw_r, int_r = _sc_gather_inter(table_flat, raw_idx, a_idx, b_idx)
    raw2 = raw_r.reshape(_B, _F * _D)
    int2 = int_r.reshape(_B, _NP)

    def fold(w, bias, g, bb):
        s = g * _BN_S
        return w * s[None, :], bias * s + bb

    w0f, b0f = fold(W0, b0, g0, bb0)
    w1f, b1f = fold(W1, b1, g1, bb1)
    w2f, b2f = fold(W2, b2, g2, bb2)
    w0t = w0f[: _F * _D]
    w0b = w0f[_F * _D:]

    out = _tc_mlp(raw2, int2, w0t, w0b, b0f[None, :], w1f, b1f[None, :],
                  w2f, b2f[None, :], W3, b3[None, :])
    return out.reshape(_B)
